# Initial kernel scaffold; baseline (speedup 1.0000x reference)
import jax
import jax.numpy as jnp
from jax.experimental import pallas as pl


def kernel(indices, prompts):
    raise NotImplementedError("write your pallas kernel here")


# speedup vs baseline: 1.1805x; 1.1805x over previous
import jax
import jax.numpy as jnp
from jax import lax
from jax.experimental import pallas as pl
from jax.experimental.pallas import tpu as pltpu
from jax.experimental.pallas import tpu_sc as plsc

_NUM_ENTRIES = 1000
_DEPTH = 12
_NUM_PER_SLOT = 4
_EMBED_DIM = 768
_ROW = _NUM_PER_SLOT * _EMBED_DIM
_B = 1024
_K = 2
_BK = _B * _K
_NC = 2
_NS = 16
_NW = _NC * _NS
_PER_W = _BK // _NW
_BLK = 16
_NBLK = _PER_W // _BLK


def _sc_body(idx_hbm, tab_hbm, out_hbm, idx_v, buf0, buf1,
             gsem0, gsem1, ssem0, ssem1):
    wid = lax.axis_index("s") * _NC + lax.axis_index("c")
    base = wid * _PER_W
    pltpu.sync_copy(idx_hbm.at[pl.ds(base, _PER_W)], idx_v)
    bufs = (buf0, buf1)
    gsems = (gsem0, gsem1)
    ssems = (ssem0, ssem1)

    def block(j, carry):
        idx12 = idx_v[pl.ds(j * _BLK, _BLK)] * _DEPTH
        row0 = base + j * _BLK
        scatters = [None, None]
        for d in range(_DEPTH):
            p = d % 2
            if scatters[p] is not None:
                scatters[p].wait()
            pltpu.async_copy(tab_hbm.at[idx12 + d], bufs[p], gsems[p]).wait()
            scatters[p] = pltpu.async_copy(
                bufs[p], out_hbm.at[d, pl.ds(row0, _BLK), :], ssems[p])
        scatters[0].wait()
        scatters[1].wait()
        return carry

    lax.fori_loop(0, _NBLK, block, 0)


_mesh = plsc.VectorSubcoreMesh(
    core_axis_name="c", subcore_axis_name="s",
    num_cores=_NC, num_subcores=_NS)

_sc_call = pl.kernel(
    _sc_body,
    out_type=jax.ShapeDtypeStruct((_DEPTH, _BK, _ROW), jnp.float32),
    mesh=_mesh,
    scratch_types=[
        pltpu.VMEM((_PER_W,), jnp.int32),
        pltpu.VMEM((_BLK, _ROW), jnp.float32),
        pltpu.VMEM((_BLK, _ROW), jnp.float32),
        pltpu.SemaphoreType.DMA,
        pltpu.SemaphoreType.DMA,
        pltpu.SemaphoreType.DMA,
        pltpu.SemaphoreType.DMA,
    ],
)


@jax.jit
def kernel(indices, prompts):
    flat = indices.reshape(-1)
    tab = prompts.reshape(_NUM_ENTRIES * _DEPTH, _ROW)
    out = _sc_call(flat, tab)
    return out.reshape(_DEPTH, _B, _K * _NUM_PER_SLOT, _EMBED_DIM)


# speedup vs baseline: 3.3311x; 2.8217x over previous
import jax
import jax.numpy as jnp
from jax import lax
from jax.experimental import pallas as pl
from jax.experimental.pallas import tpu as pltpu
from jax.experimental.pallas import tpu_sc as plsc

_NUM_ENTRIES = 1000
_DEPTH = 12
_NUM_PER_SLOT = 4
_EMBED_DIM = 768
_ROW = _NUM_PER_SLOT * _EMBED_DIM
_B = 1024
_K = 2
_BK = _B * _K
_NC = 2
_NS = 16
_NW = _NC * _NS
_PER_W = _BK // _NW
_BLK = 16
_NBLK = _PER_W // _BLK


def _sc_body(idx_hbm, tab_hbm, out4_hbm, idx_v, buf0, buf1,
             gsem0, gsem1, ssem0, ssem1):
    wid = lax.axis_index("s") * _NC + lax.axis_index("c")
    base = wid * _PER_W
    pltpu.sync_copy(idx_hbm.at[pl.ds(base, _PER_W)], idx_v)
    bufs = (buf0, buf1)
    gsems = (gsem0, gsem1)
    ssems = (ssem0, ssem1)

    def block(j, carry):
        idx12 = idx_v[pl.ds(j * _BLK, _BLK)] * _DEPTH
        b0 = (base + j * _BLK) // _K
        scatters = [None, None]
        for d in range(_DEPTH):
            p = d % 2
            if scatters[p] is not None:
                scatters[p].wait()
            pltpu.async_copy(tab_hbm.at[idx12 + d], bufs[p], gsems[p]).wait()
            scatters[p] = pltpu.async_copy(
                bufs[p].reshape(_BLK // _K, _K * _NUM_PER_SLOT, _EMBED_DIM),
                out4_hbm.at[d, pl.ds(b0, _BLK // _K), :, :], ssems[p])
        scatters[0].wait()
        scatters[1].wait()
        return carry

    lax.fori_loop(0, _NBLK, block, 0)


_mesh = plsc.VectorSubcoreMesh(
    core_axis_name="c", subcore_axis_name="s",
    num_cores=_NC, num_subcores=_NS)

_sc_call = pl.kernel(
    _sc_body,
    out_type=jax.ShapeDtypeStruct((_DEPTH, _B, _K * _NUM_PER_SLOT, _EMBED_DIM),
                                  jnp.float32),
    mesh=_mesh,
    scratch_types=[
        pltpu.VMEM((_PER_W,), jnp.int32),
        pltpu.VMEM((_BLK, _NUM_PER_SLOT, _EMBED_DIM), jnp.float32),
        pltpu.VMEM((_BLK, _NUM_PER_SLOT, _EMBED_DIM), jnp.float32),
        pltpu.SemaphoreType.DMA,
        pltpu.SemaphoreType.DMA,
        pltpu.SemaphoreType.DMA,
        pltpu.SemaphoreType.DMA,
    ],
)


@jax.jit
def kernel(indices, prompts):
    flat = indices.reshape(-1)
    tab = prompts.reshape(_NUM_ENTRIES * _DEPTH, _NUM_PER_SLOT, _EMBED_DIM)
    return _sc_call(flat, tab)
